# baseline (device time: 84279 ns/iter reference)
import jax
import jax.numpy as jnp
from jax import lax
from jax.experimental import pallas as pl
from jax.experimental.pallas import tpu as pltpu

M = 4096
N_GLOBAL = 2048
N_OUT = 1024
HALF = 512
CHUNKS = 16
M_CHUNK = M // CHUNKS


def kernel(x):
    x2d = x.reshape(M, N_GLOBAL)

    def body(x_hbm, out_ref, a_recv, send_buf, stage_send, stage_mine,
             a_send_sems, a_recv_sems, b_send_sems, b_recv_sems,
             ss_sems, sm_sems):
        my_x = lax.axis_index("x")
        my_y = lax.axis_index("y")
        peer_y = (my_x, 1 - my_y)
        peer_x = (1 - my_x, my_y)

        col_a_send = (1 - my_y) * N_OUT + my_x * HALF
        col_mine = my_y * N_OUT + my_x * HALF
        out_mine = my_x * HALF

        def start_stage(c):
            rows = pl.ds(c * M_CHUNK, M_CHUNK)
            cp_s = pltpu.make_async_copy(
                x_hbm.at[rows, pl.ds(col_a_send, HALF)],
                stage_send.at[c % 2], ss_sems.at[c % 2])
            cp_s.start()
            cp_m = pltpu.make_async_copy(
                x_hbm.at[rows, pl.ds(col_mine, HALF)],
                stage_mine.at[c % 2], sm_sems.at[c % 2])
            cp_m.start()
            return cp_s, cp_m

        stages = {c: start_stage(c) for c in range(min(2, CHUNKS))}

        barrier = pltpu.get_barrier_semaphore()
        for nbr in (peer_y, peer_x):
            pl.semaphore_signal(barrier, inc=1, device_id=nbr,
                                device_id_type=pl.DeviceIdType.MESH)
        pl.semaphore_wait(barrier, 2)

        a_descs = []
        b_descs = []
        for c in range(CHUNKS):
            rows = pl.ds(c * M_CHUNK, M_CHUNK)
            cp_s, cp_m = stages[c]

            cp_s.wait()
            send_buf[rows, :] = stage_send[c % 2].astype(jnp.bfloat16)
            d = pltpu.make_async_remote_copy(
                src_ref=send_buf.at[rows, :],
                dst_ref=a_recv.at[rows, :],
                send_sem=a_send_sems.at[c],
                recv_sem=a_recv_sems.at[c],
                device_id=peer_y,
                device_id_type=pl.DeviceIdType.MESH,
            )
            d.start()
            a_descs.append(d)

            cp_m.wait()
            a_descs[c].wait_recv()
            s = stage_mine[c % 2] + a_recv[rows, :].astype(jnp.float32)
            out_ref[rows, pl.ds(out_mine, HALF)] = s.astype(jnp.bfloat16)
            d = pltpu.make_async_remote_copy(
                src_ref=out_ref.at[rows, pl.ds(out_mine, HALF)],
                dst_ref=out_ref.at[rows, pl.ds(out_mine, HALF)],
                send_sem=b_send_sems.at[c],
                recv_sem=b_recv_sems.at[c],
                device_id=peer_x,
                device_id_type=pl.DeviceIdType.MESH,
            )
            d.start()
            b_descs.append(d)

            if c + 2 < CHUNKS:
                stages[c + 2] = start_stage(c + 2)

        for c in range(CHUNKS):
            b_descs[c].wait_recv()
        for c in range(CHUNKS):
            a_descs[c].wait_send()
            b_descs[c].wait_send()

    return pl.pallas_call(
        body,
        out_shape=jax.ShapeDtypeStruct((M, N_OUT), jnp.bfloat16),
        in_specs=[pl.BlockSpec(memory_space=pl.ANY)],
        out_specs=pl.BlockSpec(memory_space=pltpu.VMEM),
        scratch_shapes=[
            pltpu.VMEM((M, HALF), jnp.bfloat16),
            pltpu.VMEM((M, HALF), jnp.bfloat16),
            pltpu.VMEM((2, M_CHUNK, HALF), jnp.float32),
            pltpu.VMEM((2, M_CHUNK, HALF), jnp.float32),
            pltpu.SemaphoreType.DMA((CHUNKS,)),
            pltpu.SemaphoreType.DMA((CHUNKS,)),
            pltpu.SemaphoreType.DMA((CHUNKS,)),
            pltpu.SemaphoreType.DMA((CHUNKS,)),
            pltpu.SemaphoreType.DMA((2,)),
            pltpu.SemaphoreType.DMA((2,)),
        ],
        compiler_params=pltpu.CompilerParams(collective_id=0),
    )(x2d)


# device time: 62117 ns/iter; 1.3568x vs baseline; 1.3568x over previous
import jax
import jax.numpy as jnp
from jax import lax
from jax.experimental import pallas as pl
from jax.experimental.pallas import tpu as pltpu

M = 4096
N_GLOBAL = 2048
N_OUT = 1024
HALF = 512
CHUNKS = 16
M_CHUNK = M // CHUNKS


def kernel(x):
    x2d = x.reshape(M, N_GLOBAL)

    def body(x_hbm, out_ref, a_recv, send_buf, stage_send, stage_mine,
             a_send_sems, a_recv_sems, b_send_sems, b_recv_sems,
             ss_sems, sm_sems):
        my_x = lax.axis_index("x")
        my_y = lax.axis_index("y")
        peer_y = (my_x, 1 - my_y)
        peer_x = (1 - my_x, my_y)

        col_a_send = (1 - my_y) * N_OUT + my_x * HALF
        col_mine = my_y * N_OUT + my_x * HALF
        out_mine = my_x * HALF

        def start_stage_send(c):
            rows = pl.ds(c * M_CHUNK, M_CHUNK)
            cp = pltpu.make_async_copy(
                x_hbm.at[rows, pl.ds(col_a_send, HALF)],
                stage_send.at[c % 2], ss_sems.at[c % 2])
            cp.start()
            return cp

        def start_stage_mine(c):
            rows = pl.ds(c * M_CHUNK, M_CHUNK)
            cp = pltpu.make_async_copy(
                x_hbm.at[rows, pl.ds(col_mine, HALF)],
                stage_mine.at[c % 2], sm_sems.at[c % 2])
            cp.start()
            return cp

        cps = {c: start_stage_send(c) for c in range(min(2, CHUNKS))}
        cpm = {c: start_stage_mine(c) for c in range(min(2, CHUNKS))}

        barrier = pltpu.get_barrier_semaphore()
        for nbr in (peer_y, peer_x):
            pl.semaphore_signal(barrier, inc=1, device_id=nbr,
                                device_id_type=pl.DeviceIdType.MESH)
        pl.semaphore_wait(barrier, 2)

        a_descs = []
        for c in range(CHUNKS):
            rows = pl.ds(c * M_CHUNK, M_CHUNK)
            cps[c].wait()
            send_buf[rows, :] = stage_send[c % 2].astype(jnp.bfloat16)
            d = pltpu.make_async_remote_copy(
                src_ref=send_buf.at[rows, :],
                dst_ref=a_recv.at[rows, :],
                send_sem=a_send_sems.at[c],
                recv_sem=a_recv_sems.at[c],
                device_id=peer_y,
                device_id_type=pl.DeviceIdType.MESH,
            )
            d.start()
            a_descs.append(d)
            if c + 2 < CHUNKS:
                cps[c + 2] = start_stage_send(c + 2)

        b_descs = []
        for c in range(CHUNKS):
            rows = pl.ds(c * M_CHUNK, M_CHUNK)
            cpm[c].wait()
            a_descs[c].wait_recv()
            s = stage_mine[c % 2] + a_recv[rows, :].astype(jnp.float32)
            out_ref[rows, pl.ds(out_mine, HALF)] = s.astype(jnp.bfloat16)
            d = pltpu.make_async_remote_copy(
                src_ref=out_ref.at[rows, pl.ds(out_mine, HALF)],
                dst_ref=out_ref.at[rows, pl.ds(out_mine, HALF)],
                send_sem=b_send_sems.at[c],
                recv_sem=b_recv_sems.at[c],
                device_id=peer_x,
                device_id_type=pl.DeviceIdType.MESH,
            )
            d.start()
            b_descs.append(d)
            if c + 2 < CHUNKS:
                cpm[c + 2] = start_stage_mine(c + 2)

        for c in range(CHUNKS):
            b_descs[c].wait_recv()
        for c in range(CHUNKS):
            a_descs[c].wait_send()
            b_descs[c].wait_send()

    return pl.pallas_call(
        body,
        out_shape=jax.ShapeDtypeStruct((M, N_OUT), jnp.bfloat16),
        in_specs=[pl.BlockSpec(memory_space=pl.ANY)],
        out_specs=pl.BlockSpec(memory_space=pltpu.VMEM),
        scratch_shapes=[
            pltpu.VMEM((M, HALF), jnp.bfloat16),
            pltpu.VMEM((M, HALF), jnp.bfloat16),
            pltpu.VMEM((2, M_CHUNK, HALF), jnp.float32),
            pltpu.VMEM((2, M_CHUNK, HALF), jnp.float32),
            pltpu.SemaphoreType.DMA((CHUNKS,)),
            pltpu.SemaphoreType.DMA((CHUNKS,)),
            pltpu.SemaphoreType.DMA((CHUNKS,)),
            pltpu.SemaphoreType.DMA((CHUNKS,)),
            pltpu.SemaphoreType.DMA((2,)),
            pltpu.SemaphoreType.DMA((2,)),
        ],
        compiler_params=pltpu.CompilerParams(collective_id=0),
    )(x2d)


# device time: 61333 ns/iter; 1.3741x vs baseline; 1.0128x over previous
import jax
import jax.numpy as jnp
from jax import lax
from jax.experimental import pallas as pl
from jax.experimental.pallas import tpu as pltpu

M = 4096
N_GLOBAL = 2048
N_OUT = 1024
HALF = 512
CHUNKS = 8
M_CHUNK = M // CHUNKS


def kernel(x):

    def body(x_hbm, out_ref, a_recv, send_buf, mine_buf, stage_send, stage_mine,
             a_send_sems, a_recv_sems, b_send_sems, b_recv_sems,
             ss_sems, sm_sems):
        my_x = lax.axis_index("x")
        my_y = lax.axis_index("y")
        peer_y = (my_x, 1 - my_y)
        peer_x = (1 - my_x, my_y)

        col_a_send = (1 - my_y) * N_OUT + my_x * HALF
        col_mine = my_y * N_OUT + my_x * HALF
        out_mine = my_x * HALF

        def start_stage_send(c):
            rows = pl.ds(c * M_CHUNK, M_CHUNK)
            cp = pltpu.make_async_copy(
                x_hbm.at[0, rows, pl.ds(col_a_send, HALF)],
                stage_send.at[c % 2], ss_sems.at[c % 2])
            cp.start()
            return cp

        def start_stage_mine(c):
            rows = pl.ds(c * M_CHUNK, M_CHUNK)
            cp = pltpu.make_async_copy(
                x_hbm.at[0, rows, pl.ds(col_mine, HALF)],
                stage_mine.at[c % 2], sm_sems.at[c % 2])
            cp.start()
            return cp

        cps = {c: start_stage_send(c) for c in range(min(2, CHUNKS))}
        cpm = {c: start_stage_mine(c) for c in range(min(2, CHUNKS))}

        barrier = pltpu.get_barrier_semaphore()
        for nbr in (peer_y, peer_x):
            pl.semaphore_signal(barrier, inc=1, device_id=nbr,
                                device_id_type=pl.DeviceIdType.MESH)
        pl.semaphore_wait(barrier, 2)

        a_descs = []
        for c in range(CHUNKS):
            rows = pl.ds(c * M_CHUNK, M_CHUNK)
            cps[c].wait()
            send_buf[rows, :] = stage_send[c % 2].astype(jnp.bfloat16)
            d = pltpu.make_async_remote_copy(
                src_ref=send_buf.at[rows, :],
                dst_ref=a_recv.at[rows, :],
                send_sem=a_send_sems.at[c],
                recv_sem=a_recv_sems.at[c],
                device_id=peer_y,
                device_id_type=pl.DeviceIdType.MESH,
            )
            d.start()
            a_descs.append(d)
            cpm[c].wait()
            mine_buf[rows, :] = stage_mine[c % 2].astype(jnp.bfloat16)
            if c + 2 < CHUNKS:
                cps[c + 2] = start_stage_send(c + 2)
                cpm[c + 2] = start_stage_mine(c + 2)

        b_descs = []
        for c in range(CHUNKS):
            rows = pl.ds(c * M_CHUNK, M_CHUNK)
            a_descs[c].wait_recv()
            out_ref[rows, pl.ds(out_mine, HALF)] = (
                mine_buf[rows, :] + a_recv[rows, :])
            d = pltpu.make_async_remote_copy(
                src_ref=out_ref.at[rows, pl.ds(out_mine, HALF)],
                dst_ref=out_ref.at[rows, pl.ds(out_mine, HALF)],
                send_sem=b_send_sems.at[c],
                recv_sem=b_recv_sems.at[c],
                device_id=peer_x,
                device_id_type=pl.DeviceIdType.MESH,
            )
            d.start()
            b_descs.append(d)

        for c in range(CHUNKS):
            b_descs[c].wait_recv()
        for c in range(CHUNKS):
            a_descs[c].wait_send()
            b_descs[c].wait_send()

    return pl.pallas_call(
        body,
        out_shape=jax.ShapeDtypeStruct((M, N_OUT), jnp.bfloat16),
        in_specs=[pl.BlockSpec(memory_space=pl.ANY)],
        out_specs=pl.BlockSpec(memory_space=pltpu.VMEM),
        scratch_shapes=[
            pltpu.VMEM((M, HALF), jnp.bfloat16),
            pltpu.VMEM((M, HALF), jnp.bfloat16),
            pltpu.VMEM((M, HALF), jnp.bfloat16),
            pltpu.VMEM((2, M_CHUNK, HALF), jnp.float32),
            pltpu.VMEM((2, M_CHUNK, HALF), jnp.float32),
            pltpu.SemaphoreType.DMA((CHUNKS,)),
            pltpu.SemaphoreType.DMA((CHUNKS,)),
            pltpu.SemaphoreType.DMA((CHUNKS,)),
            pltpu.SemaphoreType.DMA((CHUNKS,)),
            pltpu.SemaphoreType.DMA((2,)),
            pltpu.SemaphoreType.DMA((2,)),
        ],
        compiler_params=pltpu.CompilerParams(collective_id=0),
    )(x)
